# Initial kernel scaffold; baseline (speedup 1.0000x reference)
import jax
import jax.numpy as jnp
from jax.experimental import pallas as pl


def kernel(x, edge_index, edge_attr, batch, We, Wl, eps, gamma, beta):
    raise NotImplementedError("write your pallas kernel here")


# speedup vs baseline: 2.3775x; 2.3775x over previous
import functools

import jax
import jax.numpy as jnp
from jax import lax
from jax.experimental import pallas as pl
from jax.experimental.pallas import tpu as pltpu
from jax.experimental.pallas import tpu_sc as plsc

_LANES = 16


def _edge_body(ea_ref, we_ref, out_ref):
    out_ref[0] = jnp.dot(
        ea_ref[...], we_ref[0],
        preferred_element_type=jnp.float32,
    )


def _edge_matmul(edge_attr, We):
    L, D, F = We.shape
    E = edge_attr.shape[0]
    BE = 3200
    assert E % BE == 0
    return pl.pallas_call(
        _edge_body,
        grid=(L, E // BE),
        in_specs=[
            pl.BlockSpec((BE, D), lambda l, i: (i, 0)),
            pl.BlockSpec((1, D, F), lambda l, i: (l, 0, 0)),
        ],
        out_specs=pl.BlockSpec((1, BE, F), lambda l, i: (l, i, 0)),
        out_shape=jax.ShapeDtypeStruct((L, E, F), jnp.float32),
    )(edge_attr, We)


@functools.cache
def _make_sc_layer(N, F, E):
    NW = 32
    EPW = E // NW
    K = 80
    NCH = EPW // K
    ZCH = N // K
    ZPT = (ZCH + 15) // 16
    assert EPW % K == 0 and N % K == 0 and F % _LANES == 0
    mesh = plsc.VectorSubcoreMesh(core_axis_name="c", subcore_axis_name="s")

    @functools.partial(
        pl.kernel,
        mesh=mesh,
        out_type=jax.ShapeDtypeStruct((2, N, F), jnp.float32),
        scratch_types=[
            pltpu.VMEM((K,), jnp.int32),
            pltpu.VMEM((K,), jnp.int32),
            pltpu.VMEM((K, F), jnp.float32),
            pltpu.VMEM((K, F), jnp.float32),
            pltpu.VMEM_SHARED((N, F), jnp.float32),
            pltpu.SemaphoreType.DMA,
            pltpu.SemaphoreType.DMA,
        ],
    )
    def sc_layer(x_hbm, e_hbm, src_hbm, dst_hbm, out_hbm,
                 srcv, dstv, bufe, bufx, acc, sem_e, sem_x):
        c = lax.axis_index("c")
        s = lax.axis_index("s")
        wid = c * 16 + s

        def zrow(i, _):
            for j in range(F // _LANES):
                bufe[i, pl.ds(j * _LANES, _LANES)] = jnp.zeros((_LANES,), jnp.float32)
            return 0
        lax.fori_loop(0, K, zrow, 0)

        def zchunk(i, _):
            cid = s + i * 16
            @pl.when(cid < ZCH)
            def _():
                pltpu.sync_copy(bufe, acc.at[pl.ds(cid * K, K)])
            return 0
        lax.fori_loop(0, ZPT, zchunk, 0)
        plsc.subcore_barrier()

        def chunk(t, _):
            base = wid * EPW + t * K
            pltpu.sync_copy(src_hbm.at[pl.ds(base, K)], srcv)
            pltpu.sync_copy(dst_hbm.at[pl.ds(base, K)], dstv)
            cp_e = pltpu.async_copy(e_hbm.at[pl.ds(base, K)], bufe, sem_e)
            cp_x = pltpu.async_copy(x_hbm.at[srcv], bufx, sem_x)
            cp_e.wait()
            cp_x.wait()

            def row(i, _):
                for j in range(F // _LANES):
                    sl = pl.ds(j * _LANES, _LANES)
                    bufx[i, sl] = jnp.maximum(bufx[i, sl] + bufe[i, sl], 0.0)
                return 0
            lax.fori_loop(0, K, row, 0)

            pltpu.sync_copy(bufx, acc.at[dstv], add=True)
            return 0
        lax.fori_loop(0, NCH, chunk, 0)
        plsc.subcore_barrier()

        def ochunk(i, _):
            cid = s + i * 16
            @pl.when(cid < ZCH)
            def _():
                pltpu.sync_copy(acc.at[pl.ds(cid * K, K)], bufe)
                pltpu.sync_copy(bufe, out_hbm.at[c, pl.ds(cid * K, K)])
            return 0
        lax.fori_loop(0, ZPT, ochunk, 0)

    return sc_layer


def _update_body(eps_ref, x_ref, agg_ref, wl_ref, g_ref, b_ref, out_ref):
    xs = x_ref[...] * (1.0 + eps_ref[0]) + agg_ref[0] + agg_ref[1]
    h = jnp.dot(
        xs, wl_ref[...],
        preferred_element_type=jnp.float32,
    )
    mean = jnp.mean(h, axis=0, keepdims=True)
    hc = h - mean
    var = jnp.mean(hc * hc, axis=0, keepdims=True)
    hn = hc * lax.rsqrt(var + 1e-5) * g_ref[...] + b_ref[...]
    out_ref[...] = jnp.maximum(hn, 0.0) + x_ref[...]


def _update(x, agg, wl, eps_l, gamma_l, beta_l):
    N, F = x.shape
    return pl.pallas_call(
        _update_body,
        in_specs=[
            pl.BlockSpec(memory_space=pltpu.SMEM),
            pl.BlockSpec((N, F), lambda: (0, 0)),
            pl.BlockSpec((2, N, F), lambda: (0, 0, 0)),
            pl.BlockSpec((F, F), lambda: (0, 0)),
            pl.BlockSpec((1, F), lambda: (0, 0)),
            pl.BlockSpec((1, F), lambda: (0, 0)),
        ],
        out_specs=pl.BlockSpec((N, F), lambda: (0, 0)),
        out_shape=jax.ShapeDtypeStruct((N, F), jnp.float32),
    )(eps_l.reshape(1), x, agg, wl, gamma_l.reshape(1, F), beta_l.reshape(1, F))


def kernel(x, edge_index, edge_attr, batch, We, Wl, eps, gamma, beta):
    del batch
    N, F = x.shape
    E = edge_index.shape[1]
    L = We.shape[0]
    src = edge_index[0]
    dst = edge_index[1]

    e_all = _edge_matmul(edge_attr, We)
    sc_layer = _make_sc_layer(N, F, E)

    h = x
    for l in range(L):
        agg = sc_layer(h, e_all[l], src, dst)
        h = _update(h, agg, Wl[l], eps[l], gamma[l], beta[l])
    return h


# speedup vs baseline: 3.5226x; 1.4816x over previous
import functools

import jax
import jax.numpy as jnp
from jax import lax
from jax.experimental import pallas as pl
from jax.experimental.pallas import tpu as pltpu
from jax.experimental.pallas import tpu_sc as plsc

_LANES = 16


def _edge_body(ea_ref, we_ref, out_ref):
    for l in range(we_ref.shape[0]):
        out_ref[l] = jnp.dot(
            ea_ref[...], we_ref[l],
            preferred_element_type=jnp.float32,
        )


def _edge_matmul(edge_attr, We):
    L, D, F = We.shape
    E = edge_attr.shape[0]
    BE = 6400
    assert E % BE == 0
    return pl.pallas_call(
        _edge_body,
        grid=(E // BE,),
        in_specs=[
            pl.BlockSpec((BE, D), lambda i: (i, 0)),
            pl.BlockSpec((L, D, F), lambda i: (0, 0, 0)),
        ],
        out_specs=pl.BlockSpec((L, BE, F), lambda i: (0, i, 0)),
        out_shape=jax.ShapeDtypeStruct((L, E, F), jnp.float32),
    )(edge_attr, We)


@functools.cache
def _make_sc_layer(N, F, E):
    NW = 32
    EPW = E // NW
    K = 80
    NCH = EPW // K
    ZCH = N // K
    ZPT = (ZCH + 15) // 16
    assert EPW % K == 0 and N % K == 0 and F % _LANES == 0 and NCH % 2 == 1
    mesh = plsc.VectorSubcoreMesh(core_axis_name="c", subcore_axis_name="s")

    @functools.partial(
        pl.kernel,
        mesh=mesh,
        out_type=jax.ShapeDtypeStruct((2, N, F), jnp.float32),
        scratch_types=[
            pltpu.VMEM((2, K), jnp.int32),
            pltpu.VMEM((2, K), jnp.int32),
            pltpu.VMEM((K, F), jnp.float32),
            pltpu.VMEM((K, F), jnp.float32),
            pltpu.VMEM_SHARED((N, F), jnp.float32),
            pltpu.SemaphoreType.DMA,
            pltpu.SemaphoreType.DMA,
            pltpu.SemaphoreType.DMA,
            pltpu.SemaphoreType.DMA,
            pltpu.SemaphoreType.DMA,
            pltpu.SemaphoreType.DMA,
        ],
    )
    def sc_layer(x_hbm, e_hbm, src_hbm, dst_hbm, out_hbm,
                 srcall, dstall, buf0, buf1, acc,
                 sem_e0, sem_e1, sem_g0, sem_g1, sem_s0, sem_s1):
        c = lax.axis_index("c")
        s = lax.axis_index("s")
        wid = c * 16 + s
        bufs = (buf0, buf1)
        sem_e = (sem_e0, sem_e1)
        sem_g = (sem_g0, sem_g1)
        sem_s = (sem_s0, sem_s1)

        def zrow(i, _):
            for j in range(F // _LANES):
                buf0[i, pl.ds(j * _LANES, _LANES)] = jnp.zeros((_LANES,), jnp.float32)
            return 0
        lax.fori_loop(0, K, zrow, 0)

        def zchunk(i, _):
            cid = s + i * 16
            @pl.when(cid < ZCH)
            def _():
                pltpu.sync_copy(buf0, acc.at[pl.ds(cid * K, K)])
            return 0
        lax.fori_loop(0, ZPT, zchunk, 0)

        plsc.subcore_barrier()

        def load_idx(t, b):
            base = wid * EPW + t * K
            pltpu.sync_copy(src_hbm.at[pl.ds(base, K)], srcall.at[b])
            pltpu.sync_copy(dst_hbm.at[pl.ds(base, K)], dstall.at[b])

        def issue_e(t, b):
            return pltpu.async_copy(
                e_hbm.at[pl.ds(wid * EPW + t * K, K)], bufs[b], sem_e[b])

        def wait_e(b):
            pltpu.make_async_copy(
                e_hbm.at[pl.ds(0, K)], bufs[b], sem_e[b]).wait()

        def issue_gather(b):
            pltpu.async_copy(x_hbm.at[srcall.at[b]], bufs[b], sem_g[b],
                             add=True)

        def wait_gather(b):
            pltpu.make_async_copy(
                x_hbm.at[srcall.at[b]], bufs[b], sem_g[b]).wait()

        def wait_scatter(b):
            pltpu.make_async_copy(bufs[b], acc.at[dstall.at[b]], sem_s[b]).wait()

        def relu_buf(b):
            def row(i, _):
                for j in range(F // _LANES):
                    sl = pl.ds(j * _LANES, _LANES)
                    bufs[b][i, sl] = jnp.maximum(bufs[b][i, sl], 0.0)
                return 0
            lax.fori_loop(0, K, row, 0)

        def step(t, b, first=False, last=False):
            nb = 1 - b
            if not last:
                if not first:
                    wait_scatter(nb)
                load_idx(t + 1, nb)
                issue_e(t + 1, nb)
            wait_gather(b)
            relu_buf(b)
            pltpu.async_copy(bufs[b], acc.at[dstall.at[b]], sem_s[b], add=True)
            if not last:
                wait_e(nb)
                issue_gather(nb)

        load_idx(0, 0)
        issue_e(0, 0)
        wait_e(0)
        issue_gather(0)

        step(0, 0, first=True)

        def pair(p, _):
            t0 = 2 * p + 1
            step(t0, 1)
            step(t0 + 1, 0)
            return 0
        lax.fori_loop(0, (NCH - 3) // 2, pair, 0)
        step(NCH - 2, 1)
        step(NCH - 1, 0, last=True)

        wait_scatter(0)
        wait_scatter(1)
        plsc.subcore_barrier()

        def ochunk(i, _):
            cid = s + i * 16
            @pl.when(cid < ZCH)
            def _():
                pltpu.sync_copy(acc.at[pl.ds(cid * K, K)], buf0)
                pltpu.sync_copy(buf0, out_hbm.at[c, pl.ds(cid * K, K)])
            return 0
        lax.fori_loop(0, ZPT, ochunk, 0)

    return sc_layer


def _update_body(eps_ref, x_ref, agg_ref, wl_ref, g_ref, b_ref, out_ref):
    xs = x_ref[...] * (1.0 + eps_ref[0]) + agg_ref[0] + agg_ref[1]
    h = jnp.dot(
        xs, wl_ref[...],
        preferred_element_type=jnp.float32,
    )
    mean = jnp.mean(h, axis=0, keepdims=True)
    hc = h - mean
    var = jnp.mean(hc * hc, axis=0, keepdims=True)
    hn = hc * lax.rsqrt(var + 1e-5) * g_ref[...] + b_ref[...]
    out_ref[...] = jnp.maximum(hn, 0.0) + x_ref[...]


def _update(x, agg, wl, eps_l, gamma_l, beta_l):
    N, F = x.shape
    return pl.pallas_call(
        _update_body,
        in_specs=[
            pl.BlockSpec(memory_space=pltpu.SMEM),
            pl.BlockSpec((N, F), lambda: (0, 0)),
            pl.BlockSpec((2, N, F), lambda: (0, 0, 0)),
            pl.BlockSpec((F, F), lambda: (0, 0)),
            pl.BlockSpec((1, F), lambda: (0, 0)),
            pl.BlockSpec((1, F), lambda: (0, 0)),
        ],
        out_specs=pl.BlockSpec((N, F), lambda: (0, 0)),
        out_shape=jax.ShapeDtypeStruct((N, F), jnp.float32),
    )(eps_l.reshape(1), x, agg, wl, gamma_l.reshape(1, F), beta_l.reshape(1, F))


def kernel(x, edge_index, edge_attr, batch, We, Wl, eps, gamma, beta):
    del batch
    N, F = x.shape
    E = edge_index.shape[1]
    L = We.shape[0]
    src = edge_index[0]
    dst = edge_index[1]

    e_all = _edge_matmul(edge_attr, We)
    sc_layer = _make_sc_layer(N, F, E)

    h = x
    for l in range(L):
        agg = sc_layer(h, e_all[l], src, dst)
        h = _update(h, agg, Wl[l], eps[l], gamma[l], beta[l])
    return h


# speedup vs baseline: 4.7469x; 1.3476x over previous
import functools

import jax
import jax.numpy as jnp
from jax import lax
from jax.experimental import pallas as pl
from jax.experimental.pallas import tpu as pltpu
from jax.experimental.pallas import tpu_sc as plsc

_LANES = 16


def _edge_body(ea_ref, we_ref, *out_refs):
    for l, out_ref in enumerate(out_refs):
        out_ref[...] = jnp.dot(
            ea_ref[...], we_ref[l],
            preferred_element_type=jnp.float32,
        )


def _edge_matmul(edge_attr, We):
    L, D, F = We.shape
    E = edge_attr.shape[0]
    BE = 6400
    assert E % BE == 0
    return pl.pallas_call(
        _edge_body,
        grid=(E // BE,),
        in_specs=[
            pl.BlockSpec((BE, D), lambda i: (i, 0)),
            pl.BlockSpec((L, D, F), lambda i: (0, 0, 0)),
        ],
        out_specs=[pl.BlockSpec((BE, F), lambda i: (i, 0))] * L,
        out_shape=[jax.ShapeDtypeStruct((E, F), jnp.float32)] * L,
    )(edge_attr, We)


@functools.cache
def _make_sc_layer(N, F, E):
    NW = 32
    EPW = E // NW
    K = 80
    NB = 4
    NCH = EPW // K
    ZCH = N // K
    ZPT = (ZCH + 15) // 16
    assert EPW % K == 0 and N % K == 0 and F % _LANES == 0
    assert NCH % NB == 1 and NCH > NB
    mesh = plsc.VectorSubcoreMesh(core_axis_name="c", subcore_axis_name="s")

    @functools.partial(
        pl.kernel,
        mesh=mesh,
        out_type=jax.ShapeDtypeStruct((2, N, F), jnp.float32),
        scratch_types=[
            pltpu.VMEM((NB, K), jnp.int32),
            pltpu.VMEM((NB, K), jnp.int32),
        ] + [pltpu.VMEM((K, F), jnp.float32)] * NB
          + [pltpu.VMEM_SHARED((N, F), jnp.float32)]
          + [pltpu.SemaphoreType.DMA] * (4 * NB),
    )
    def sc_layer(x_hbm, e_hbm, src_hbm, dst_hbm, out_hbm,
                 srcall, dstall, *bufs_acc_sems):
        bufs = bufs_acc_sems[:NB]
        acc = bufs_acc_sems[NB]
        sems = bufs_acc_sems[NB + 1:]
        sem_e = sems[0:NB]
        sem_g = sems[NB:2 * NB]
        sem_s = sems[2 * NB:3 * NB]
        sem_i = sems[3 * NB:4 * NB]
        c = lax.axis_index("c")
        s = lax.axis_index("s")
        wid = c * 16 + s

        def zrow(i, _):
            for j in range(F // _LANES):
                bufs[0][i, pl.ds(j * _LANES, _LANES)] = jnp.zeros(
                    (_LANES,), jnp.float32)
            return 0
        lax.fori_loop(0, K, zrow, 0)

        def zchunk(i, _):
            cid = s + i * 16
            @pl.when(cid < ZCH)
            def _():
                pltpu.sync_copy(bufs[0], acc.at[pl.ds(cid * K, K)])
            return 0
        lax.fori_loop(0, ZPT, zchunk, 0)
        plsc.subcore_barrier()

        def issue_idx(t, b):
            base = wid * EPW + t * K
            pltpu.async_copy(src_hbm.at[pl.ds(base, K)], srcall.at[b], sem_i[b])
            pltpu.async_copy(dst_hbm.at[pl.ds(base, K)], dstall.at[b], sem_i[b])

        def wait_idx(b):
            pltpu.make_async_copy(
                src_hbm.at[pl.ds(0, K)], srcall.at[b], sem_i[b]).wait()
            pltpu.make_async_copy(
                dst_hbm.at[pl.ds(0, K)], dstall.at[b], sem_i[b]).wait()

        def issue_e(t, b):
            pltpu.async_copy(
                e_hbm.at[pl.ds(wid * EPW + t * K, K)], bufs[b], sem_e[b])

        def wait_e(b):
            pltpu.make_async_copy(
                e_hbm.at[pl.ds(0, K)], bufs[b], sem_e[b]).wait()

        def wait_scatter(b):
            pltpu.make_async_copy(bufs[b], acc.at[dstall.at[b]], sem_s[b]).wait()

        def relu_buf(b):
            def row(i, _):
                for j in range(F // _LANES):
                    sl = pl.ds(j * _LANES, _LANES)
                    bufs[b][i, sl] = jnp.maximum(bufs[b][i, sl], 0.0)
                return 0
            lax.fori_loop(0, K, row, 0)

        def prep(tn, nb, guard_ws):
            if guard_ws:
                wait_scatter(nb)
            issue_idx(tn, nb)
            issue_e(tn, nb)

        def chain(nb):
            wait_e(nb)
            wait_idx(nb)
            pltpu.async_copy(x_hbm.at[srcall.at[nb]], bufs[nb], sem_g[nb],
                             add=True)

        def step(t, b, first=False, last=False):
            nb = (b + 1) % NB
            if not last:
                prep(t + 1, nb, guard_ws=not first)
            pltpu.make_async_copy(
                x_hbm.at[srcall.at[b]], bufs[b], sem_g[b]).wait()
            relu_buf(b)
            pltpu.async_copy(bufs[b], acc.at[dstall.at[b]], sem_s[b], add=True)
            if not last:
                chain(nb)

        issue_idx(0, 0)
        issue_e(0, 0)
        chain(0)
        for t in range(NB - 1):
            step(t, t, first=True)

        def group(p, _):
            t0 = NB * p + NB - 1
            for j in range(NB):
                step(t0 + j, (NB - 1 + j) % NB)
            return 0
        lax.fori_loop(0, (NCH - NB) // NB, group, 0)
        step(NCH - 2, (NCH - 2) % NB)
        step(NCH - 1, (NCH - 1) % NB, last=True)

        for b in range(NB):
            wait_scatter(b)
        plsc.subcore_barrier()

        def ochunk(i, _):
            cid = s + i * 16
            @pl.when(cid < ZCH)
            def _():
                pltpu.sync_copy(acc.at[pl.ds(cid * K, K)], bufs[0])
                pltpu.sync_copy(bufs[0], out_hbm.at[c, pl.ds(cid * K, K)])
            return 0
        lax.fori_loop(0, ZPT, ochunk, 0)

    return sc_layer


def _update_body(eps_ref, x_ref, agg_ref, wl_ref, g_ref, b_ref, out_ref):
    xs = x_ref[...] * (1.0 + eps_ref[0]) + agg_ref[0] + agg_ref[1]
    h = jnp.dot(
        xs, wl_ref[...],
        preferred_element_type=jnp.float32,
    )
    mean = jnp.mean(h, axis=0, keepdims=True)
    hc = h - mean
    var = jnp.mean(hc * hc, axis=0, keepdims=True)
    hn = hc * lax.rsqrt(var + 1e-5) * g_ref[...] + b_ref[...]
    out_ref[...] = jnp.maximum(hn, 0.0) + x_ref[...]


def _update(x, agg, wl, eps_l, gamma_l, beta_l):
    N, F = x.shape
    return pl.pallas_call(
        _update_body,
        in_specs=[
            pl.BlockSpec(memory_space=pltpu.SMEM),
            pl.BlockSpec((N, F), lambda: (0, 0)),
            pl.BlockSpec((2, N, F), lambda: (0, 0, 0)),
            pl.BlockSpec((F, F), lambda: (0, 0)),
            pl.BlockSpec((1, F), lambda: (0, 0)),
            pl.BlockSpec((1, F), lambda: (0, 0)),
        ],
        out_specs=pl.BlockSpec((N, F), lambda: (0, 0)),
        out_shape=jax.ShapeDtypeStruct((N, F), jnp.float32),
    )(eps_l.reshape(1), x, agg, wl, gamma_l.reshape(1, F), beta_l.reshape(1, F))


def kernel(x, edge_index, edge_attr, batch, We, Wl, eps, gamma, beta):
    del batch
    N, F = x.shape
    E = edge_index.shape[1]
    L = We.shape[0]
    src = edge_index[0]
    dst = edge_index[1]

    e_all = _edge_matmul(edge_attr, We)
    sc_layer = _make_sc_layer(N, F, E)

    h = x
    for l in range(L):
        agg = sc_layer(h, e_all[l], src, dst)
        h = _update(h, agg, Wl[l], eps[l], gamma[l], beta[l])
    return h


# speedup vs baseline: 5.2203x; 1.0997x over previous
import functools

import jax
import jax.numpy as jnp
from jax import lax
from jax.experimental import pallas as pl
from jax.experimental.pallas import tpu as pltpu
from jax.experimental.pallas import tpu_sc as plsc

_LANES = 16


def _edge_body(ea_ref, we_ref, *out_refs):
    for l, out_ref in enumerate(out_refs):
        out_ref[...] = jnp.dot(
            ea_ref[...], we_ref[l],
            preferred_element_type=jnp.float32,
        )


def _edge_matmul(edge_attr, We):
    L, D, F = We.shape
    E = edge_attr.shape[0]
    BE = 6400
    assert E % BE == 0
    return pl.pallas_call(
        _edge_body,
        grid=(E // BE,),
        in_specs=[
            pl.BlockSpec((BE, D), lambda i: (i, 0)),
            pl.BlockSpec((L, D, F), lambda i: (0, 0, 0)),
        ],
        out_specs=[pl.BlockSpec((BE, F), lambda i: (i, 0))] * L,
        out_shape=[jax.ShapeDtypeStruct((E, F), jnp.float32)] * L,
    )(edge_attr, We)


@functools.cache
def _make_sc_layer(N, F, E):
    NW = 32
    EPW = E // NW
    K = 80
    NB = 4
    NCH = EPW // K
    ZCH = N // K
    ZPT = (ZCH + 15) // 16
    assert EPW % K == 0 and N % K == 0 and F % _LANES == 0
    assert (NCH - 5) % NB == 0 and NCH > NB + 2
    mesh = plsc.VectorSubcoreMesh(core_axis_name="c", subcore_axis_name="s")

    @functools.partial(
        pl.kernel,
        mesh=mesh,
        out_type=jax.ShapeDtypeStruct((2, N, F), jnp.float32),
        scratch_types=[
            pltpu.VMEM((NB, K), jnp.int32),
            pltpu.VMEM((NB, K), jnp.int32),
        ] + [pltpu.VMEM((K, F), jnp.float32)] * NB
          + [pltpu.VMEM_SHARED((N, F), jnp.float32)]
          + [pltpu.SemaphoreType.DMA] * (4 * NB),
    )
    def sc_layer(x_hbm, e_hbm, src_hbm, dst_hbm, out_hbm,
                 srcall, dstall, *bufs_acc_sems):
        bufs = bufs_acc_sems[:NB]
        acc = bufs_acc_sems[NB]
        sems = bufs_acc_sems[NB + 1:]
        sem_e = sems[0:NB]
        sem_g = sems[NB:2 * NB]
        sem_s = sems[2 * NB:3 * NB]
        sem_i = sems[3 * NB:4 * NB]
        c = lax.axis_index("c")
        s = lax.axis_index("s")
        wid = c * 16 + s

        def zrow(i, _):
            for j in range(F // _LANES):
                bufs[0][i, pl.ds(j * _LANES, _LANES)] = jnp.zeros(
                    (_LANES,), jnp.float32)
            return 0
        lax.fori_loop(0, K, zrow, 0)

        def zchunk(i, _):
            cid = s + i * 16
            @pl.when(cid < ZCH)
            def _():
                pltpu.sync_copy(bufs[0], acc.at[pl.ds(cid * K, K)])
            return 0
        lax.fori_loop(0, ZPT, zchunk, 0)
        plsc.subcore_barrier()

        def issue_idx(t, b):
            base = wid * EPW + t * K
            pltpu.async_copy(src_hbm.at[pl.ds(base, K)], srcall.at[b], sem_i[b])
            pltpu.async_copy(dst_hbm.at[pl.ds(base, K)], dstall.at[b], sem_i[b])

        def wait_idx(b):
            pltpu.make_async_copy(
                src_hbm.at[pl.ds(0, K)], srcall.at[b], sem_i[b]).wait()
            pltpu.make_async_copy(
                dst_hbm.at[pl.ds(0, K)], dstall.at[b], sem_i[b]).wait()

        def issue_e(t, b):
            pltpu.async_copy(
                e_hbm.at[pl.ds(wid * EPW + t * K, K)], bufs[b], sem_e[b])

        def wait_e(b):
            pltpu.make_async_copy(
                e_hbm.at[pl.ds(0, K)], bufs[b], sem_e[b]).wait()

        def wait_scatter(b):
            pltpu.make_async_copy(bufs[b], acc.at[dstall.at[b]], sem_s[b]).wait()

        def relu_buf(b):
            def row(i, _):
                for j in range(F // _LANES):
                    sl = pl.ds(j * _LANES, _LANES)
                    bufs[b][i, sl] = jnp.maximum(bufs[b][i, sl], 0.0)
                return 0
            lax.fori_loop(0, K, row, 0)

        def chain(nb):
            wait_e(nb)
            wait_idx(nb)
            pltpu.async_copy(x_hbm.at[srcall.at[nb]], bufs[nb], sem_g[nb],
                             add=True)

        def step(t, b, s1=True, s2=True, ws=True):
            g = (b + 1) % NB
            pe = (b + 2) % NB
            if s1:
                chain(g)
            if s2:
                if ws:
                    wait_scatter(pe)
                issue_idx(t + 2, pe)
                issue_e(t + 2, pe)
            pltpu.make_async_copy(
                x_hbm.at[srcall.at[b]], bufs[b], sem_g[b]).wait()
            relu_buf(b)
            pltpu.async_copy(bufs[b], acc.at[dstall.at[b]], sem_s[b], add=True)

        issue_idx(0, 0)
        issue_e(0, 0)
        issue_idx(1, 1)
        issue_e(1, 1)
        chain(0)
        step(0, 0, ws=False)
        step(1, 1, ws=False)

        def group(p, _):
            t0 = NB * p + 2
            for j in range(NB):
                step(t0 + j, (2 + j) % NB)
            return 0
        lax.fori_loop(0, (NCH - 5) // NB, group, 0)
        step(NCH - 3, (NCH - 3) % NB)
        step(NCH - 2, (NCH - 2) % NB, s2=False)
        step(NCH - 1, (NCH - 1) % NB, s1=False, s2=False)

        for b in range(NB):
            wait_scatter(b)
        plsc.subcore_barrier()

        def ochunk(i, _):
            cid = s + i * 16
            @pl.when(cid < ZCH)
            def _():
                pltpu.sync_copy(acc.at[pl.ds(cid * K, K)], bufs[0])
                pltpu.sync_copy(bufs[0], out_hbm.at[c, pl.ds(cid * K, K)])
            return 0
        lax.fori_loop(0, ZPT, ochunk, 0)

    return sc_layer


def _update_body(eps_ref, x_ref, agg_ref, wl_ref, g_ref, b_ref, out_ref):
    xs = x_ref[...] * (1.0 + eps_ref[0]) + agg_ref[0] + agg_ref[1]
    h = jnp.dot(
        xs, wl_ref[...],
        preferred_element_type=jnp.float32,
    )
    mean = jnp.mean(h, axis=0, keepdims=True)
    hc = h - mean
    var = jnp.mean(hc * hc, axis=0, keepdims=True)
    hn = hc * lax.rsqrt(var + 1e-5) * g_ref[...] + b_ref[...]
    out_ref[...] = jnp.maximum(hn, 0.0) + x_ref[...]


def _update(x, agg, wl, eps_l, gamma_l, beta_l):
    N, F = x.shape
    return pl.pallas_call(
        _update_body,
        in_specs=[
            pl.BlockSpec(memory_space=pltpu.SMEM),
            pl.BlockSpec((N, F), lambda: (0, 0)),
            pl.BlockSpec((2, N, F), lambda: (0, 0, 0)),
            pl.BlockSpec((F, F), lambda: (0, 0)),
            pl.BlockSpec((1, F), lambda: (0, 0)),
            pl.BlockSpec((1, F), lambda: (0, 0)),
        ],
        out_specs=pl.BlockSpec((N, F), lambda: (0, 0)),
        out_shape=jax.ShapeDtypeStruct((N, F), jnp.float32),
    )(eps_l.reshape(1), x, agg, wl, gamma_l.reshape(1, F), beta_l.reshape(1, F))


def kernel(x, edge_index, edge_attr, batch, We, Wl, eps, gamma, beta):
    del batch
    N, F = x.shape
    E = edge_index.shape[1]
    L = We.shape[0]
    src = edge_index[0]
    dst = edge_index[1]

    e_all = _edge_matmul(edge_attr, We)
    sc_layer = _make_sc_layer(N, F, E)

    h = x
    for l in range(L):
        agg = sc_layer(h, e_all[l], src, dst)
        h = _update(h, agg, Wl[l], eps[l], gamma[l], beta[l])
    return h


# speedup vs baseline: 6.6014x; 1.2646x over previous
import functools

import jax
import jax.numpy as jnp
import numpy as np
from jax import lax
from jax.experimental import pallas as pl
from jax.experimental.pallas import tpu as pltpu
from jax.experimental.pallas import tpu_sc as plsc

_LANES = 16


def _perm_lo(F):
    w = np.arange(F // 2)
    return 32 * (w // 16) + (w % 16)


def _pack_words(lo, hi):
    lo_u = jax.lax.bitcast_convert_type(
        lo.astype(jnp.bfloat16), jnp.uint16).astype(jnp.uint32)
    hi_u = jax.lax.bitcast_convert_type(
        hi.astype(jnp.bfloat16), jnp.uint16).astype(jnp.uint32)
    return jax.lax.bitcast_convert_type(lo_u | (hi_u << 16), jnp.int32)


def _edge_body(ea_a_ref, ea_b_ref, wlo_ref, whi_ref, *out_refs):
    for l, out_ref in enumerate(out_refs):
        pa = _pack_words(
            jnp.dot(ea_a_ref[...], wlo_ref[l],
                    preferred_element_type=jnp.float32),
            jnp.dot(ea_a_ref[...], whi_ref[l],
                    preferred_element_type=jnp.float32))
        pb = _pack_words(
            jnp.dot(ea_b_ref[...], wlo_ref[l],
                    preferred_element_type=jnp.float32),
            jnp.dot(ea_b_ref[...], whi_ref[l],
                    preferred_element_type=jnp.float32))
        out_ref[...] = jnp.concatenate([pa, pb], axis=1)


def _edge_matmul(edge_attr, We_lo, We_hi):
    L, D, H = We_lo.shape
    E = edge_attr.shape[0]
    BE = 6400
    NBLK = E // (2 * BE)
    assert E % (2 * BE) == 0
    return pl.pallas_call(
        _edge_body,
        grid=(NBLK,),
        in_specs=[
            pl.BlockSpec((BE, D), lambda i: (i, 0)),
            pl.BlockSpec((BE, D), lambda i: (i + NBLK, 0)),
            pl.BlockSpec((L, D, H), lambda i: (0, 0, 0)),
            pl.BlockSpec((L, D, H), lambda i: (0, 0, 0)),
        ],
        out_specs=[pl.BlockSpec((BE, 2 * H), lambda i: (i, 0))] * L,
        out_shape=[jax.ShapeDtypeStruct((E // 2, 2 * H), jnp.int32)] * L,
    )(edge_attr, edge_attr, We_lo, We_hi)


@functools.cache
def _make_sc_layer(N, F, E):
    NW = 32
    EPW = E // NW
    K = 80
    HK = K // 2
    HE = E // 2
    HEPW = EPW // 2
    NB = 3
    NCH = EPW // K
    ZCH = N // K
    ZPT = (ZCH + 15) // 16
    assert EPW % K == 0 and N % K == 0 and F % 32 == 0 and HK % 8 == 0
    assert (NCH - 5) % NB == 0 and NCH > NB + 2
    mesh = plsc.VectorSubcoreMesh(core_axis_name="c", subcore_axis_name="s")

    @functools.partial(
        pl.kernel,
        mesh=mesh,
        out_type=jax.ShapeDtypeStruct((2, N, F), jnp.float32),
        scratch_types=[
            pltpu.VMEM((NB, K), jnp.int32),
            pltpu.VMEM((NB, K), jnp.int32),
        ] + [pltpu.VMEM((HK, F), jnp.int32)] * NB
          + [pltpu.VMEM((K, F), jnp.float32)] * NB
          + [pltpu.VMEM_SHARED((N, F), jnp.float32)]
          + [pltpu.SemaphoreType.DMA] * (4 * NB),
    )
    def sc_layer(x_hbm, e_hbm, src_hbm, dst_hbm, out_hbm,
                 srcall, dstall, *bufs_acc_sems):
        ebufs = bufs_acc_sems[:NB]
        xbufs = bufs_acc_sems[NB:2 * NB]
        acc = bufs_acc_sems[2 * NB]
        sems = bufs_acc_sems[2 * NB + 1:]
        sem_e = sems[0:NB]
        sem_g = sems[NB:2 * NB]
        sem_s = sems[2 * NB:3 * NB]
        sem_i = sems[3 * NB:4 * NB]
        c = lax.axis_index("c")
        s = lax.axis_index("s")
        wid = c * 16 + s

        zbuf = xbufs[0]
        def zrow(i, _):
            for j in range(F // _LANES):
                zbuf[i, pl.ds(j * _LANES, _LANES)] = jnp.zeros(
                    (_LANES,), jnp.float32)
            return 0
        lax.fori_loop(0, K, zrow, 0)

        def zchunk(i, _):
            cid = s + i * 16
            @pl.when(cid < ZCH)
            def _():
                pltpu.sync_copy(zbuf, acc.at[pl.ds(cid * K, K)])
            return 0
        lax.fori_loop(0, ZPT, zchunk, 0)
        plsc.subcore_barrier()

        def issue_idx(t, b):
            base = (wid * NCH + t) * K
            pltpu.async_copy(src_hbm.at[pl.ds(base, K)], srcall.at[b], sem_i[b])
            pltpu.async_copy(dst_hbm.at[pl.ds(base, K)], dstall.at[b], sem_i[b])

        def wait_idx(b):
            pltpu.make_async_copy(
                src_hbm.at[pl.ds(0, K)], srcall.at[b], sem_i[b]).wait()
            pltpu.make_async_copy(
                dst_hbm.at[pl.ds(0, K)], dstall.at[b], sem_i[b]).wait()

        def issue_e(t, b):
            pltpu.async_copy(
                e_hbm.at[pl.ds(wid * HEPW + t * HK, HK)], ebufs[b], sem_e[b])

        def wait_e(b):
            pltpu.make_async_copy(
                e_hbm.at[pl.ds(0, HK)], ebufs[b], sem_e[b]).wait()

        def wait_scatter(b):
            pltpu.make_async_copy(
                xbufs[b], acc.at[dstall.at[b]], sem_s[b]).wait()

        def compute(b):
            eb, xb = ebufs[b], xbufs[b]
            def row(q, _):
                for half in range(2):
                    r = q + half * HK
                    for blk in range(F // 32):
                        ew = eb[q, pl.ds(half * (F // 2) + blk * _LANES,
                                         _LANES)]
                        e0 = jax.lax.bitcast_convert_type(ew << 16, jnp.float32)
                        e1 = jax.lax.bitcast_convert_type(ew & jnp.int32(-65536), jnp.float32)
                        s0 = pl.ds(blk * 32, _LANES)
                        s1 = pl.ds(blk * 32 + _LANES, _LANES)
                        xb[r, s0] = jnp.maximum(xb[r, s0] + e0, 0.0)
                        xb[r, s1] = jnp.maximum(xb[r, s1] + e1, 0.0)
                return 0
            lax.fori_loop(0, HK, row, 0)

        def step(t, b, s1=True, s2=True, ws=True):
            g = (b + 1) % NB
            pe = (b + 2) % NB
            if s1:
                wait_idx(g)
                pltpu.async_copy(x_hbm.at[srcall.at[g]], xbufs[g], sem_g[g])
            if s2:
                if ws:
                    wait_scatter(pe)
                issue_idx(t + 2, pe)
                issue_e(t + 2, pe)
            wait_e(b)
            pltpu.make_async_copy(
                x_hbm.at[srcall.at[b]], xbufs[b], sem_g[b]).wait()
            compute(b)
            pltpu.async_copy(xbufs[b], acc.at[dstall.at[b]], sem_s[b], add=True)

        issue_idx(0, 0)
        issue_e(0, 0)
        issue_idx(1, 1)
        issue_e(1, 1)
        wait_idx(0)
        pltpu.async_copy(x_hbm.at[srcall.at[0]], xbufs[0], sem_g[0])
        step(0, 0, ws=False)
        step(1, 1)

        def group(p, _):
            t0 = NB * p + 2
            for j in range(NB):
                step(t0 + j, (2 + j) % NB)
            return 0
        lax.fori_loop(0, (NCH - 5) // NB, group, 0)
        step(NCH - 3, (NCH - 3) % NB)
        step(NCH - 2, (NCH - 2) % NB, s2=False)
        step(NCH - 1, (NCH - 1) % NB, s1=False, s2=False)

        for b in range(NB):
            wait_scatter(b)
        plsc.subcore_barrier()

        def ochunk(i, _):
            cid = s + i * 16
            @pl.when(cid < ZCH)
            def _():
                pltpu.sync_copy(acc.at[pl.ds(cid * K, K)], zbuf)
                pltpu.sync_copy(zbuf, out_hbm.at[c, pl.ds(cid * K, K)])
            return 0
        lax.fori_loop(0, ZPT, ochunk, 0)

    return sc_layer


def _update_body(eps_ref, x_ref, agg_ref, wl_ref, g_ref, b_ref, out_ref):
    xs = x_ref[...] * (1.0 + eps_ref[0]) + agg_ref[0] + agg_ref[1]
    h = jnp.dot(xs, wl_ref[...], preferred_element_type=jnp.float32)
    mean = jnp.mean(h, axis=0, keepdims=True)
    hc = h - mean
    var = jnp.mean(hc * hc, axis=0, keepdims=True)
    hn = hc * lax.rsqrt(var + 1e-5) * g_ref[...] + b_ref[...]
    out_ref[...] = jnp.maximum(hn, 0.0) + x_ref[...]


def _update(x, agg, wl, eps_l, gamma_l, beta_l):
    N, F = x.shape
    return pl.pallas_call(
        _update_body,
        in_specs=[pl.BlockSpec(memory_space=pltpu.SMEM)]
        + [pl.BlockSpec(memory_space=pltpu.VMEM)] * 5,
        out_shape=jax.ShapeDtypeStruct((N, F), jnp.float32),
    )(eps_l.reshape(1), x, agg, wl, gamma_l.reshape(1, F), beta_l.reshape(1, F))


def kernel(x, edge_index, edge_attr, batch, We, Wl, eps, gamma, beta):
    del batch
    N, F = x.shape
    E = edge_index.shape[1]
    L = We.shape[0]
    HK = 40
    def pair_idx(v):
        return jnp.concatenate(
            [v[:E // 2].reshape(-1, HK), v[E // 2:].reshape(-1, HK)],
            axis=1).reshape(-1)
    src = pair_idx(edge_index[0])
    dst = pair_idx(edge_index[1])

    plo = _perm_lo(F)
    We_lo = We[:, :, plo]
    We_hi = We[:, :, plo + 16]

    e_all = _edge_matmul(edge_attr, We_lo, We_hi)
    sc_layer = _make_sc_layer(N, F, E)

    h = x
    for l in range(L):
        agg = sc_layer(h, e_all[l], src, dst)
        h = _update(h, agg, Wl[l], eps[l], gamma[l], beta[l])
    return h
